# baseline (speedup 1.0000x reference)
import functools

import jax
import jax.numpy as jnp
from jax import lax
from jax.experimental import pallas as pl

TOPK_K = 200
LANES = 128


def _score_stage(det_ref, sc_ref, cls_ref, box_ref):
    d = det_ref[0]
    C, F = d.shape
    lane = lax.broadcasted_iota(jnp.int32, (C, F), 1)
    neg = jnp.float32(-jnp.inf)
    dm = jnp.where(lane >= 5, d, neg)
    m = jnp.max(dm, axis=1, keepdims=True)
    am = jnp.min(jnp.where(dm == m, lane - 5, jnp.int32(F)), axis=1,
                 keepdims=True)
    obj = jnp.max(jnp.where(lane == 4, d, neg), axis=1, keepdims=True)
    sc_ref[0] = m * obj
    cls_ref[0] = am.astype(jnp.float32)
    box_ref[0] = d[:, 0:4]


def _topk_stage(sp_ref, cls_ref, box_ref, out_ref, *, k, n_pad, h, w):
    s0 = sp_ref[0]
    R = s0.shape[0]
    row = lax.broadcasted_iota(jnp.int32, (R, LANES), 0)
    col = lax.broadcasted_iota(jnp.int32, (R, LANES), 1)
    flat = row * LANES + col
    b_f = pl.program_id(0).astype(jnp.float32)
    neg = jnp.float32(-jnp.inf)
    lane4 = lax.broadcasted_iota(jnp.int32, (1, 4), 1)
    coef = jnp.where(lane4 < 2, jnp.float32(-0.5), jnp.float32(0.5))
    scale = jnp.where(lane4 % 2 == 0, jnp.float32(h - 1.0),
                      jnp.float32(w - 1.0))

    def body(i, s):
        m = jnp.max(s)
        idx = jnp.min(jnp.where(s == m, flat, jnp.int32(n_pad)))
        s2 = jnp.where(flat == idx, neg, s)
        cv = cls_ref[0, pl.ds(idx, 1), :]
        bv = box_ref[0, pl.ds(idx, 1), :]
        base = jnp.concatenate([bv[:, 0:2], bv[:, 0:2]], axis=1)
        wh = jnp.concatenate([bv[:, 2:4], bv[:, 2:4]], axis=1)
        corners = (base + wh * coef) / scale
        bm = jnp.zeros((1, 1), jnp.float32) + b_f
        mm = jnp.zeros((1, 1), jnp.float32) + m
        row = jnp.concatenate([bm, cv, mm, corners], axis=1)
        out_ref[0, pl.ds(i, 1), :] = row
        return s2

    lax.fori_loop(0, k, body, s0)


def kernel(x, det):
    B, N, F = det.shape
    H, W = x.shape[2], x.shape[3]
    K = TOPK_K
    C = 2000
    NC = N // C

    scores3, cls3, boxes = pl.pallas_call(
        _score_stage,
        grid=(B, NC),
        in_specs=[pl.BlockSpec((1, C, F), lambda b, i: (b, i, 0))],
        out_specs=[
            pl.BlockSpec((1, C, 1), lambda b, i: (b, i, 0)),
            pl.BlockSpec((1, C, 1), lambda b, i: (b, i, 0)),
            pl.BlockSpec((1, C, 4), lambda b, i: (b, i, 0)),
        ],
        out_shape=[
            jax.ShapeDtypeStruct((B, N, 1), jnp.float32),
            jax.ShapeDtypeStruct((B, N, 1), jnp.float32),
            jax.ShapeDtypeStruct((B, N, 4), jnp.float32),
        ],
    )(det)

    n_pad = ((N + LANES - 1) // LANES) * LANES
    R = n_pad // LANES
    sp = jnp.pad(scores3.reshape(B, N), ((0, 0), (0, n_pad - N)),
                 constant_values=-jnp.inf).reshape(B, R, LANES)

    final = pl.pallas_call(
        functools.partial(_topk_stage, k=K, n_pad=n_pad,
                          h=float(H), w=float(W)),
        grid=(B,),
        in_specs=[
            pl.BlockSpec((1, R, LANES), lambda b: (b, 0, 0)),
            pl.BlockSpec((1, N, 1), lambda b: (b, 0, 0)),
            pl.BlockSpec((1, N, 4), lambda b: (b, 0, 0)),
        ],
        out_specs=pl.BlockSpec((1, K, 7), lambda b: (b, 0, 0)),
        out_shape=jax.ShapeDtypeStruct((B, K, 7), jnp.float32),
    )(sp, cls3, boxes)

    final = final.reshape(B * K, 7)
    return (final, final[:, 0])


# speedup vs baseline: 1.8995x; 1.8995x over previous
import functools

import jax
import jax.numpy as jnp
from jax import lax
from jax.experimental import pallas as pl
from jax.experimental.pallas import tpu as pltpu
from jax.experimental.pallas import tpu_sc as plsc

TOPK_K = 200
LANES = 128
REC = 8
RPR = LANES // REC


def _score_stage(det_ref, sc_ref, cls_ref, box_ref):
    d = det_ref[0]
    C, F = d.shape
    lane = lax.broadcasted_iota(jnp.int32, (C, F), 1)
    neg = jnp.float32(-jnp.inf)
    dm = jnp.where(lane >= 5, d, neg)
    m = jnp.max(dm, axis=1, keepdims=True)
    am = jnp.min(jnp.where(dm == m, lane - 5, jnp.int32(F)), axis=1,
                 keepdims=True)
    obj = jnp.max(jnp.where(lane == 4, d, neg), axis=1, keepdims=True)
    sc_ref[0] = m * obj
    cls_ref[0] = am.astype(jnp.float32)
    box_ref[0] = d[:, 0:4]


def _select_stage(sp_ref, idx_ref, s_scratch, *, k, n_pad, kp):
    B = sp_ref.shape[0]
    s_scratch[...] = sp_ref[...]
    col = lax.broadcasted_iota(jnp.int32, (B, n_pad), 1)
    kcol = lax.broadcasted_iota(jnp.int32, (B, kp), 1)
    neg = jnp.float32(-jnp.inf)

    def body(i, acc):
        s = s_scratch[...]
        m = jnp.max(s, axis=1, keepdims=True)
        idx = jnp.min(jnp.where(s == m, col, jnp.int32(n_pad)),
                      axis=1, keepdims=True)
        s_scratch[...] = jnp.where(col == idx, neg, s)
        return jnp.where(kcol == i, idx, acc)

    idx_ref[...] = lax.fori_loop(0, k, body,
                                 jnp.zeros((B, kp), jnp.int32))


def _gather_stage(idx_hbm, rec_hbm, out_hbm, idx_v, sub_v, rows_v, outc_v,
                  sem, *, n, kw, h, w):
    cid = lax.axis_index("c")
    sid = lax.axis_index("s")
    wid = sid * 2 + cid
    b = wid // 2
    pltpu.sync_copy(idx_hbm.at[wid], idx_v)
    bbase = b * n

    nchunk = kw // 16
    for t in range(nchunk):
        sl = pl.ds(t * 16, 16)
        g = idx_v[sl] + bbase
        idx_v[sl] = jax.lax.shift_right_logical(g, 4)
        sub_v[sl] = jnp.bitwise_and(g, 15)
    pltpu.async_copy(rec_hbm.at[idx_v], rows_v, sem).wait()

    b_f = b.astype(jnp.float32)
    lane = lax.broadcasted_iota(jnp.int32, (16,), 0)

    for t in range(nchunk):
        subv = sub_v[pl.ds(t * 16, 16)]
        for p2 in range(16):
            p = t * 16 + p2
            s = subv[p2]
            off = jax.lax.shift_left(
                jax.lax.shift_right_logical(s, 1), 4)
            e_hi = jnp.bitwise_and(s, 1) == 1
            ch = rows_v[p, pl.ds(off, 16)]
            cx = jnp.where(e_hi, ch[8], ch[0])
            cy = jnp.where(e_hi, ch[9], ch[1])
            wv = jnp.where(e_hi, ch[10], ch[2])
            hv = jnp.where(e_hi, ch[11], ch[3])
            score = jnp.where(e_hi, ch[12], ch[4])
            cls = jnp.where(e_hi, ch[13], ch[5])
            hw = wv * jnp.float32(0.5)
            hh = hv * jnp.float32(0.5)
            vals = [
                b_f,
                cls,
                score,
                (cx - hw) * jnp.float32(1.0 / (h - 1.0)),
                (cy - hh) * jnp.float32(1.0 / (w - 1.0)),
                (cx + hw) * jnp.float32(1.0 / (h - 1.0)),
                (cy + hh) * jnp.float32(1.0 / (w - 1.0)),
            ]
            out_row = jnp.zeros((16,), jnp.float32)
            for c, v in enumerate(vals):
                out_row = jnp.where(lane == c, v, out_row)
            outc_v[p, pl.ds(0, 16)] = out_row

    pltpu.sync_copy(outc_v, out_hbm.at[wid])


def kernel(x, det):
    B, N, F = det.shape
    H, W = x.shape[2], x.shape[3]
    K = TOPK_K
    KP = 256
    KW = 112
    NR = N // RPR
    C = 2000
    NC = N // C

    scores3, cls3, boxes = pl.pallas_call(
        _score_stage,
        grid=(B, NC),
        in_specs=[pl.BlockSpec((1, C, F), lambda b, i: (b, i, 0))],
        out_specs=[
            pl.BlockSpec((1, C, 1), lambda b, i: (b, i, 0)),
            pl.BlockSpec((1, C, 1), lambda b, i: (b, i, 0)),
            pl.BlockSpec((1, C, 4), lambda b, i: (b, i, 0)),
        ],
        out_shape=[
            jax.ShapeDtypeStruct((B, N, 1), jnp.float32),
            jax.ShapeDtypeStruct((B, N, 1), jnp.float32),
            jax.ShapeDtypeStruct((B, N, 4), jnp.float32),
        ],
    )(det)

    rec = jnp.concatenate(
        [boxes, scores3, cls3, scores3, scores3], axis=2)
    rec = rec.reshape(B * NR, LANES)

    n_pad = ((N + LANES - 1) // LANES) * LANES
    sp = jnp.pad(scores3.reshape(B, N), ((0, 0), (0, n_pad - N)),
                 constant_values=-jnp.inf)

    idx = pl.pallas_call(
        functools.partial(_select_stage, k=K, n_pad=n_pad, kp=KP),
        in_specs=[pl.BlockSpec((B, n_pad), lambda: (0, 0))],
        out_specs=pl.BlockSpec((B, KP), lambda: (0, 0)),
        out_shape=jax.ShapeDtypeStruct((B, KP), jnp.int32),
        scratch_shapes=[pltpu.VMEM((B, n_pad), jnp.float32)],
    )(sp)

    idx2 = jnp.stack([idx[:, :KW], idx[:, K - KW:K]], axis=1)
    idx2 = idx2.reshape(2 * B, KW)

    mesh = plsc.VectorSubcoreMesh(core_axis_name="c", subcore_axis_name="s")
    gather = pl.kernel(
        functools.partial(_gather_stage, n=N, kw=KW,
                          h=float(H), w=float(W)),
        mesh=mesh,
        compiler_params=pltpu.CompilerParams(needs_layout_passes=False),
        out_type=jax.ShapeDtypeStruct((2 * B, KW, 16), jnp.float32),
        scratch_types=[
            pltpu.VMEM((KW,), jnp.int32),
            pltpu.VMEM((KW,), jnp.int32),
            pltpu.VMEM((KW, LANES), jnp.float32),
            pltpu.VMEM((KW, 16), jnp.float32),
            pltpu.SemaphoreType.DMA,
        ],
    )
    tiles = gather(idx2, rec)

    half = K // 2
    even = tiles[0::2, :half, :7]
    odd = tiles[1::2, half - (K - KW):KW, :7]
    final = jnp.concatenate([even, odd], axis=1).reshape(B * K, 7)
    return (final, final[:, 0])


# speedup vs baseline: 1.9278x; 1.0149x over previous
import functools

import jax
import jax.numpy as jnp
from jax import lax
from jax.experimental import pallas as pl
from jax.experimental.pallas import tpu as pltpu
from jax.experimental.pallas import tpu_sc as plsc

TOPK_K = 200
LANES = 128
REC = 8
RPR = LANES // REC


def _score_stage(det_ref, sc_ref, cls_ref, box_ref):
    d = det_ref[0]
    C, F = d.shape
    cls_part = d[:, 5:F]
    lane = lax.broadcasted_iota(jnp.int32, (C, F - 5), 1)
    lane_f = lane.astype(jnp.float32)
    m = jnp.max(cls_part, axis=1, keepdims=True)
    am = jnp.min(jnp.where(cls_part == m, lane_f, jnp.float32(F)),
                 axis=1, keepdims=True)
    obj = d[:, 4:5]
    sc_ref[0] = m * obj
    cls_ref[0] = am
    box_ref[0] = d[:, 0:4]


def _select_stage(sp_ref, idx_ref, s_scratch, *, k, n_pad, kp):
    B = sp_ref.shape[0]
    s_scratch[...] = sp_ref[...]
    col = lax.broadcasted_iota(jnp.int32, (B, n_pad), 1)
    kcol = lax.broadcasted_iota(jnp.int32, (B, kp), 1)
    neg = jnp.float32(-jnp.inf)

    def body(i, acc):
        s = s_scratch[...]
        m = jnp.max(s, axis=1, keepdims=True)
        idx = jnp.min(jnp.where(s == m, col, jnp.int32(n_pad)),
                      axis=1, keepdims=True)
        s_scratch[...] = jnp.where(col == idx, neg, s)
        return jnp.where(kcol == i, idx, acc)

    idx_ref[...] = lax.fori_loop(0, k, body,
                                 jnp.zeros((B, kp), jnp.int32))


def _gather_stage(idx_hbm, rec_hbm, out_hbm, idx_v, sub_v, rows_v, outc_v,
                  sem, *, n, kw, h, w):
    cid = lax.axis_index("c")
    sid = lax.axis_index("s")
    wid = sid * 2 + cid
    b = wid // 2
    pltpu.sync_copy(idx_hbm.at[wid], idx_v)
    bbase = b * n

    nchunk = kw // 16
    for t in range(nchunk):
        sl = pl.ds(t * 16, 16)
        g = idx_v[sl] + bbase
        idx_v[sl] = jax.lax.shift_right_logical(g, 4)
        sub_v[sl] = jnp.bitwise_and(g, 15)
    pltpu.async_copy(rec_hbm.at[idx_v], rows_v, sem).wait()

    b_f = b.astype(jnp.float32)
    lane = lax.broadcasted_iota(jnp.int32, (16,), 0)

    for t in range(nchunk):
        subv = sub_v[pl.ds(t * 16, 16)]
        for p2 in range(16):
            p = t * 16 + p2
            s = subv[p2]
            off = jax.lax.shift_left(
                jax.lax.shift_right_logical(s, 1), 4)
            e_hi = jnp.bitwise_and(s, 1) == 1
            ch = rows_v[p, pl.ds(off, 16)]
            cx = jnp.where(e_hi, ch[8], ch[0])
            cy = jnp.where(e_hi, ch[9], ch[1])
            wv = jnp.where(e_hi, ch[10], ch[2])
            hv = jnp.where(e_hi, ch[11], ch[3])
            score = jnp.where(e_hi, ch[12], ch[4])
            cls = jnp.where(e_hi, ch[13], ch[5])
            hw = wv * jnp.float32(0.5)
            hh = hv * jnp.float32(0.5)
            vals = [
                b_f,
                cls,
                score,
                (cx - hw) * jnp.float32(1.0 / (h - 1.0)),
                (cy - hh) * jnp.float32(1.0 / (w - 1.0)),
                (cx + hw) * jnp.float32(1.0 / (h - 1.0)),
                (cy + hh) * jnp.float32(1.0 / (w - 1.0)),
            ]
            out_row = jnp.zeros((16,), jnp.float32)
            for c, v in enumerate(vals):
                out_row = jnp.where(lane == c, v, out_row)
            outc_v[p, pl.ds(0, 16)] = out_row

    pltpu.sync_copy(outc_v, out_hbm.at[wid])


def kernel(x, det):
    B, N, F = det.shape
    H, W = x.shape[2], x.shape[3]
    K = TOPK_K
    KP = 256
    KW = 112
    NR = N // RPR
    C = 2000
    NC = N // C

    scores3, cls3, boxes = pl.pallas_call(
        _score_stage,
        grid=(B, NC),
        in_specs=[pl.BlockSpec((1, C, F), lambda b, i: (b, i, 0))],
        out_specs=[
            pl.BlockSpec((1, C, 1), lambda b, i: (b, i, 0)),
            pl.BlockSpec((1, C, 1), lambda b, i: (b, i, 0)),
            pl.BlockSpec((1, C, 4), lambda b, i: (b, i, 0)),
        ],
        out_shape=[
            jax.ShapeDtypeStruct((B, N, 1), jnp.float32),
            jax.ShapeDtypeStruct((B, N, 1), jnp.float32),
            jax.ShapeDtypeStruct((B, N, 4), jnp.float32),
        ],
    )(det)

    rec = jnp.concatenate(
        [boxes, scores3, cls3, scores3, scores3], axis=2)
    rec = rec.reshape(B * NR, LANES)

    n_pad = ((N + LANES - 1) // LANES) * LANES
    sp = jnp.pad(scores3.reshape(B, N), ((0, 0), (0, n_pad - N)),
                 constant_values=-jnp.inf)

    idx = pl.pallas_call(
        functools.partial(_select_stage, k=K, n_pad=n_pad, kp=KP),
        in_specs=[pl.BlockSpec((B, n_pad), lambda: (0, 0))],
        out_specs=pl.BlockSpec((B, KP), lambda: (0, 0)),
        out_shape=jax.ShapeDtypeStruct((B, KP), jnp.int32),
        scratch_shapes=[pltpu.VMEM((B, n_pad), jnp.float32)],
    )(sp)

    idx2 = jnp.stack([idx[:, :KW], idx[:, K - KW:K]], axis=1)
    idx2 = idx2.reshape(2 * B, KW)

    mesh = plsc.VectorSubcoreMesh(core_axis_name="c", subcore_axis_name="s")
    gather = pl.kernel(
        functools.partial(_gather_stage, n=N, kw=KW,
                          h=float(H), w=float(W)),
        mesh=mesh,
        compiler_params=pltpu.CompilerParams(needs_layout_passes=False),
        out_type=jax.ShapeDtypeStruct((2 * B, KW, 16), jnp.float32),
        scratch_types=[
            pltpu.VMEM((KW,), jnp.int32),
            pltpu.VMEM((KW,), jnp.int32),
            pltpu.VMEM((KW, LANES), jnp.float32),
            pltpu.VMEM((KW, 16), jnp.float32),
            pltpu.SemaphoreType.DMA,
        ],
    )
    tiles = gather(idx2, rec)

    half = K // 2
    even = tiles[0::2, :half, :7]
    odd = tiles[1::2, half - (K - KW):KW, :7]
    final = jnp.concatenate([even, odd], axis=1).reshape(B * K, 7)
    return (final, final[:, 0])


# speedup vs baseline: 3.9868x; 2.0681x over previous
import functools

import jax
import jax.numpy as jnp
from jax import lax
from jax.experimental import pallas as pl
from jax.experimental.pallas import tpu as pltpu
from jax.experimental.pallas import tpu_sc as plsc

TOPK_K = 200
LANES = 128
REC = 8
RPR = LANES // REC
SEL = 224
GCH = 13


def _score_stage(det_ref, rec_ref):
    d = det_ref[0]
    C, F = d.shape
    lane = lax.broadcasted_iota(jnp.int32, (C, F), 1)
    lane_f = (lane - 5).astype(jnp.float32)
    neg = jnp.float32(-jnp.inf)
    dm = jnp.where(lane >= 5, d, neg)
    m = jnp.max(dm, axis=1, keepdims=True)
    am = jnp.min(jnp.where(dm == m, lane_f, jnp.float32(F)),
                 axis=1, keepdims=True)
    obj = jnp.max(jnp.where(lane == 4, d, neg), axis=1, keepdims=True)
    score = m * obj
    rec8 = jnp.concatenate(
        [d[:, 0:4], score, am, score, score], axis=1)
    rec_ref[0] = rec8.T


def _thresh_stage(sp_ref, thr_ref, *, k):
    s = sp_ref[...]
    B = s.shape[0]
    si = lax.bitcast_convert_type(s, jnp.int32)

    def body(i, t):
        bit = 30 - i
        tt = jnp.bitwise_or(t, lax.shift_left(jnp.int32(1), bit))
        cnt = jnp.sum((si >= tt).astype(jnp.int32), axis=1, keepdims=True)
        return jnp.where(cnt >= k, tt, t)

    t = lax.fori_loop(0, 31, body, jnp.zeros((B, 1), jnp.int32))
    thr_ref[...] = jnp.broadcast_to(t, thr_ref.shape)


def _tree16(vs, op):
    while len(vs) > 1:
        vs = [op(vs[i], vs[i + 1]) if i + 1 < len(vs) else vs[i]
              for i in range(0, len(vs), 2)]
    return vs[0]


def _select_gather(sp_hbm, thr_hbm, rec_hbm, out_hbm, sv, thr_v, gv_ref,
                   gi_ref, ei_ref, selrow, sub_v, ri1, ri2, rows1, rows2,
                   outv, sem, *, n, np_, k, nr, h, w):
    cid = lax.axis_index("c")
    sid = lax.axis_index("s")
    wid = sid * 2 + cid
    B = out_hbm.shape[0]
    lane = lax.broadcasted_iota(jnp.int32, (16,), 0)
    MIN = jnp.int32(-2**31)
    BIG = jnp.int32(1 << 30)

    @pl.when(wid < B)
    def _():
        b = wid
        pltpu.sync_copy(sp_hbm.at[b], sv)
        pltpu.sync_copy(thr_hbm.at[b], thr_v)
        t = thr_v[pl.ds(0, 16)][0]

        for i in range(SEL // 16):
            sl = pl.ds(i * 16, 16)
            gv_ref[sl] = jnp.zeros((16,), jnp.int32) + MIN
            gi_ref[sl] = jnp.zeros((16,), jnp.int32)

        def compact(i, carry):
            cg, ce = carry
            v = sv[pl.ds(i * 16, 16)]
            vi = plsc.bitcast(v, jnp.int32)
            idxv = i * 16 + lane
            m_g = vi > t
            m_e = vi == t
            plsc.store_compressed(gv_ref.at[pl.ds(cg, 16)], vi, mask=m_g)
            plsc.store_compressed(gi_ref.at[pl.ds(cg, 16)], idxv, mask=m_g)
            plsc.store_compressed(ei_ref.at[pl.ds(ce, 16)], idxv, mask=m_e)
            cg = cg + plsc.all_reduce_population_count(m_g)[0]
            ce = ce + plsc.all_reduce_population_count(m_e)[0]
            return cg, ce

        k1, _ = lax.fori_loop(0, np_ // 16, compact,
                              (jnp.int32(0), jnp.int32(0)))

        gvs = [gv_ref[pl.ds(i * 16, 16)] for i in range(GCH)]
        gis = [gi_ref[pl.ds(i * 16, 16)] for i in range(GCH)]

        def extract(j, carry):
            gvs, gis, acc = carry
            m = lax.reduce_max(_tree16(gvs, jnp.maximum), axes=(0,))
            cand = [jnp.where(gvs[i] == m, gis[i], BIG)
                    for i in range(GCH)]
            gi = lax.reduce_min(_tree16(cand, jnp.minimum), axes=(0,))
            gvs = [jnp.where((gvs[i] == m) & (gis[i] == gi), MIN, gvs[i])
                   for i in range(GCH)]
            acc = jnp.where(lane == jnp.bitwise_and(j, 15), gi, acc)
            selrow[pl.ds(jnp.bitwise_and(j, ~15), 16)] = acc
            return gvs, gis, acc

        lax.fori_loop(0, k, extract,
                      (gvs, gis, jnp.zeros((16,), jnp.int32)))

        for i in range(GCH):
            @pl.when(k1 + i * 16 < 208)
            def _():
                selrow[pl.ds(k1 + i * 16, 16)] = ei_ref[pl.ds(i * 16, 16)]

        for i in range(SEL // 16):
            sl = pl.ds(i * 16, 16)
            rv = selrow[sl]
            rv = jnp.minimum(jnp.maximum(rv, 0), n - 1)
            selrow[sl] = rv
            rid = b * nr + lax.shift_right_logical(rv, 4)
            sub_v[sl] = jnp.bitwise_and(rv, 15)
            if i < 8:
                ri1[pl.ds(i * 16, 16)] = rid
            else:
                ri2[pl.ds((i - 8) * 16, 16)] = rid

        cp1 = pltpu.async_copy(rec_hbm.at[ri1], rows1, sem)
        cp1.wait()
        cp2 = pltpu.async_copy(rec_hbm.at[ri2], rows2, sem)
        cp2.wait()

        b_f = b.astype(jnp.float32)

        def unpack(rows_v, t, pbase):
            subv = sub_v[pl.ds(t * 16 + pbase * 16, 16)]
            for p2 in range(16):
                p = t * 16 + p2
                s = subv[p2]
                off = lax.shift_left(lax.shift_right_logical(s, 1), 4)
                e_hi = jnp.bitwise_and(s, 1) == 1
                ch = rows_v[p, pl.ds(off, 16)]
                cx = jnp.where(e_hi, ch[8], ch[0])
                cy = jnp.where(e_hi, ch[9], ch[1])
                wv = jnp.where(e_hi, ch[10], ch[2])
                hv = jnp.where(e_hi, ch[11], ch[3])
                score = jnp.where(e_hi, ch[12], ch[4])
                cls = jnp.where(e_hi, ch[13], ch[5])
                hw = wv * jnp.float32(0.5)
                hh = hv * jnp.float32(0.5)
                vals = [
                    b_f,
                    cls,
                    score,
                    (cx - hw) * jnp.float32(1.0 / (h - 1.0)),
                    (cy - hh) * jnp.float32(1.0 / (w - 1.0)),
                    (cx + hw) * jnp.float32(1.0 / (h - 1.0)),
                    (cy + hh) * jnp.float32(1.0 / (w - 1.0)),
                ]
                out_row = jnp.zeros((16,), jnp.float32)
                for c, v in enumerate(vals):
                    out_row = jnp.where(lane == c, v, out_row)
                outv[pbase * 16 + p, pl.ds(0, 16)] = out_row

        def un1(t, _):
            unpack(rows1, t, 0)
            return 0

        def un2(t, _):
            unpack(rows2, t, 8)
            return 0

        lax.fori_loop(0, 8, un1, 0)
        lax.fori_loop(0, SEL // 16 - 8, un2, 0)
        pltpu.sync_copy(outv, out_hbm.at[b])


def kernel(x, det):
    B, N, F = det.shape
    H, W = x.shape[2], x.shape[3]
    K = TOPK_K
    NR = N // RPR
    C = 2000
    NC = N // C

    rec_t = pl.pallas_call(
        _score_stage,
        grid=(B,),
        in_specs=[pl.BlockSpec((1, N, F), lambda b: (b, 0, 0))],
        out_specs=pl.BlockSpec((1, REC, N), lambda b: (b, 0, 0)),
        out_shape=jax.ShapeDtypeStruct((B, REC, N), jnp.float32),
    )(det)

    rec = rec_t.reshape(B, REC, NR, RPR).transpose(0, 2, 3, 1)
    rec = rec.reshape(B * NR, LANES)

    n_pad = ((N + LANES - 1) // LANES) * LANES
    sp = jnp.pad(rec_t[:, 4, :], ((0, 0), (0, n_pad - N)),
                 constant_values=-jnp.inf)

    thr = pl.pallas_call(
        functools.partial(_thresh_stage, k=K),
        in_specs=[pl.BlockSpec((B, n_pad), lambda: (0, 0))],
        out_specs=pl.BlockSpec((B, LANES), lambda: (0, 0)),
        out_shape=jax.ShapeDtypeStruct((B, LANES), jnp.int32),
    )(sp)

    mesh = plsc.VectorSubcoreMesh(core_axis_name="c", subcore_axis_name="s")
    select_gather = pl.kernel(
        functools.partial(_select_gather, n=N, np_=n_pad, k=K, nr=NR,
                          h=float(H), w=float(W)),
        mesh=mesh,
        compiler_params=pltpu.CompilerParams(needs_layout_passes=False),
        out_type=jax.ShapeDtypeStruct((B, SEL, 16), jnp.float32),
        scratch_types=[
            pltpu.VMEM((n_pad,), jnp.float32),
            pltpu.VMEM((LANES,), jnp.int32),
            pltpu.VMEM((SEL,), jnp.int32),
            pltpu.VMEM((SEL,), jnp.int32),
            pltpu.VMEM((n_pad,), jnp.int32),
            pltpu.VMEM((SEL,), jnp.int32),
            pltpu.VMEM((SEL,), jnp.int32),
            pltpu.VMEM((128,), jnp.int32),
            pltpu.VMEM((96,), jnp.int32),
            pltpu.VMEM((128, LANES), jnp.float32),
            pltpu.VMEM((96, LANES), jnp.float32),
            pltpu.VMEM((SEL, 16), jnp.float32),
            pltpu.SemaphoreType.DMA,
        ],
    )
    tiles = select_gather(sp, thr, rec)

    final = tiles[:, :K, :7].reshape(B * K, 7)
    return (final, final[:, 0])


# speedup vs baseline: 4.0412x; 1.0136x over previous
import functools

import jax
import jax.numpy as jnp
from jax import lax
from jax.experimental import pallas as pl
from jax.experimental.pallas import tpu as pltpu
from jax.experimental.pallas import tpu_sc as plsc

TOPK_K = 200
LANES = 128
REC = 8
RPR = LANES // REC
SEL = 224
GCH = 13


def _score_stage(det_ref, rec_ref):
    d = det_ref[0]
    C, F = d.shape
    lane = lax.broadcasted_iota(jnp.int32, (C, F), 1)
    lane_f = (lane - 5).astype(jnp.float32)
    neg = jnp.float32(-jnp.inf)
    dm = jnp.where(lane >= 5, d, neg)
    m = jnp.max(dm, axis=1, keepdims=True)
    am = jnp.min(jnp.where(dm == m, lane_f, jnp.float32(F)),
                 axis=1, keepdims=True)
    obj = jnp.max(jnp.where(lane == 4, d, neg), axis=1, keepdims=True)
    score = m * obj
    rec8 = jnp.concatenate(
        [d[:, 0:4], score, am, score, score], axis=1)
    rec_ref[0] = rec8.T


def _thresh_stage(sp_ref, thr_ref, *, k):
    s = sp_ref[...]
    B = s.shape[0]
    si = lax.bitcast_convert_type(s, jnp.int32)

    def body(i, t):
        bit = 30 - i
        tt = jnp.bitwise_or(t, lax.shift_left(jnp.int32(1), bit))
        cnt = jnp.sum((si >= tt).astype(jnp.int32), axis=1, keepdims=True)
        return jnp.where(cnt >= k, tt, t)

    t = lax.fori_loop(0, 31, body, jnp.zeros((B, 1), jnp.int32))
    thr_ref[...] = jnp.broadcast_to(t, thr_ref.shape)


def _tree16(vs, op):
    while len(vs) > 1:
        vs = [op(vs[i], vs[i + 1]) if i + 1 < len(vs) else vs[i]
              for i in range(0, len(vs), 2)]
    return vs[0]


def _select_gather(sp_hbm, thr_hbm, rec_hbm, out_hbm, sv, thr_v, gv_ref,
                   gi_ref, ei_ref, selrow, sub_v, ri1, ri2, rows1, rows2,
                   outv, sem, *, n, np_, k, nr, h, w):
    cid = lax.axis_index("c")
    sid = lax.axis_index("s")
    wid = sid * 2 + cid
    B = out_hbm.shape[0]
    lane = lax.broadcasted_iota(jnp.int32, (16,), 0)
    MIN = jnp.int32(-2**31)
    BIG = jnp.int32(1 << 30)

    @pl.when(wid < B)
    def _():
        b = wid
        pltpu.sync_copy(sp_hbm.at[b], sv)
        pltpu.sync_copy(thr_hbm.at[b], thr_v)
        t = thr_v[pl.ds(0, 16)][0]

        for i in range(SEL // 16):
            sl = pl.ds(i * 16, 16)
            gv_ref[sl] = jnp.zeros((16,), jnp.int32) + MIN
            gi_ref[sl] = jnp.zeros((16,), jnp.int32)

        def compact(i, carry):
            cg, ce = carry
            v = sv[pl.ds(i * 16, 16)]
            vi = plsc.bitcast(v, jnp.int32)
            idxv = i * 16 + lane
            m_g = vi > t
            m_e = vi == t
            plsc.store_compressed(gv_ref.at[pl.ds(cg, 16)], vi, mask=m_g)
            plsc.store_compressed(gi_ref.at[pl.ds(cg, 16)], idxv, mask=m_g)
            plsc.store_compressed(ei_ref.at[pl.ds(ce, 16)], idxv, mask=m_e)
            cg = cg + plsc.all_reduce_population_count(m_g)[0]
            ce = ce + plsc.all_reduce_population_count(m_e)[0]
            return cg, ce

        k1, _ = lax.fori_loop(0, np_ // 16, compact,
                              (jnp.int32(0), jnp.int32(0)))

        gvs = [gv_ref[pl.ds(i * 16, 16)] for i in range(GCH)]
        gis = [gi_ref[pl.ds(i * 16, 16)] for i in range(GCH)]

        def extract(j, carry):
            gvs, gis, acc = carry
            m = lax.reduce_max(_tree16(gvs, jnp.maximum), axes=(0,))
            cand = [jnp.where(gvs[i] == m, gis[i], BIG)
                    for i in range(GCH)]
            gi = lax.reduce_min(_tree16(cand, jnp.minimum), axes=(0,))
            gvs = [jnp.where((gvs[i] == m) & (gis[i] == gi), MIN, gvs[i])
                   for i in range(GCH)]
            acc = jnp.where(lane == jnp.bitwise_and(j, 15), gi, acc)
            selrow[pl.ds(jnp.bitwise_and(j, ~15), 16)] = acc
            return gvs, gis, acc

        lax.fori_loop(0, k, extract,
                      (gvs, gis, jnp.zeros((16,), jnp.int32)))

        for i in range(GCH):
            @pl.when(k1 + i * 16 < 208)
            def _():
                selrow[pl.ds(k1 + i * 16, 16)] = ei_ref[pl.ds(i * 16, 16)]

        for i in range(SEL // 16):
            sl = pl.ds(i * 16, 16)
            rv = selrow[sl]
            rv = jnp.minimum(jnp.maximum(rv, 0), n - 1)
            selrow[sl] = rv
            rid = b * nr + lax.shift_right_logical(rv, 4)
            sub_v[sl] = jnp.bitwise_and(rv, 15)
            if i < 8:
                ri1[pl.ds(i * 16, 16)] = rid
            else:
                ri2[pl.ds((i - 8) * 16, 16)] = rid

        cp1 = pltpu.async_copy(rec_hbm.at[ri1], rows1, sem)
        cp1.wait()
        cp2 = pltpu.async_copy(rec_hbm.at[ri2], rows2, sem)
        cp2.wait()

        b_f = b.astype(jnp.float32)
        bvec = jnp.zeros((16,), jnp.float32) + b_f

        def unpack(rows_v, t, pbase):
            sub = sub_v[pl.ds((t + pbase) * 16, 16)]
            rloc = t * 16 + lane

            def fld(c):
                return plsc.load_gather(rows_v, [rloc, c * 16 + sub])

            cx = fld(0)
            cy = fld(1)
            wv = fld(2)
            hv = fld(3)
            score = fld(4)
            cls = fld(5)
            hw = wv * jnp.float32(0.5)
            hh = hv * jnp.float32(0.5)
            vals = [
                bvec,
                cls,
                score,
                (cx - hw) * jnp.float32(1.0 / (h - 1.0)),
                (cy - hh) * jnp.float32(1.0 / (w - 1.0)),
                (cx + hw) * jnp.float32(1.0 / (h - 1.0)),
                (cy + hh) * jnp.float32(1.0 / (w - 1.0)),
            ]
            grow = (t + pbase) * 16 + lane
            for c2, v in enumerate(vals):
                colv = jnp.zeros((16,), jnp.int32) + c2
                plsc.store_scatter(outv, [grow, colv], v)

        def un1(t, _):
            unpack(rows1, t, 0)
            return 0

        def un2(t, _):
            unpack(rows2, t, 8)
            return 0

        lax.fori_loop(0, 8, un1, 0)
        lax.fori_loop(0, SEL // 16 - 8, un2, 0)
        pltpu.sync_copy(outv, out_hbm.at[b])


def kernel(x, det):
    B, N, F = det.shape
    H, W = x.shape[2], x.shape[3]
    K = TOPK_K
    NR = N // RPR
    C = 2000
    NC = N // C

    rec_t = pl.pallas_call(
        _score_stage,
        grid=(B,),
        in_specs=[pl.BlockSpec((1, N, F), lambda b: (b, 0, 0))],
        out_specs=pl.BlockSpec((1, REC, N), lambda b: (b, 0, 0)),
        out_shape=jax.ShapeDtypeStruct((B, REC, N), jnp.float32),
    )(det)

    rec = rec_t.reshape(B, REC, NR, RPR).transpose(0, 2, 1, 3)
    rec = rec.reshape(B * NR, LANES)

    n_pad = ((N + LANES - 1) // LANES) * LANES
    sp = jnp.pad(rec_t[:, 4, :], ((0, 0), (0, n_pad - N)),
                 constant_values=-jnp.inf)

    thr = pl.pallas_call(
        functools.partial(_thresh_stage, k=K),
        in_specs=[pl.BlockSpec((B, n_pad), lambda: (0, 0))],
        out_specs=pl.BlockSpec((B, LANES), lambda: (0, 0)),
        out_shape=jax.ShapeDtypeStruct((B, LANES), jnp.int32),
    )(sp)

    mesh = plsc.VectorSubcoreMesh(core_axis_name="c", subcore_axis_name="s")
    select_gather = pl.kernel(
        functools.partial(_select_gather, n=N, np_=n_pad, k=K, nr=NR,
                          h=float(H), w=float(W)),
        mesh=mesh,
        compiler_params=pltpu.CompilerParams(needs_layout_passes=False),
        out_type=jax.ShapeDtypeStruct((B, SEL, 16), jnp.float32),
        scratch_types=[
            pltpu.VMEM((n_pad,), jnp.float32),
            pltpu.VMEM((LANES,), jnp.int32),
            pltpu.VMEM((SEL,), jnp.int32),
            pltpu.VMEM((SEL,), jnp.int32),
            pltpu.VMEM((n_pad,), jnp.int32),
            pltpu.VMEM((SEL,), jnp.int32),
            pltpu.VMEM((SEL,), jnp.int32),
            pltpu.VMEM((128,), jnp.int32),
            pltpu.VMEM((96,), jnp.int32),
            pltpu.VMEM((128, LANES), jnp.float32),
            pltpu.VMEM((96, LANES), jnp.float32),
            pltpu.VMEM((SEL, 16), jnp.float32),
            pltpu.SemaphoreType.DMA,
        ],
    )
    tiles = select_gather(sp, thr, rec)

    final = tiles[:, :K, :7].reshape(B * K, 7)
    return (final, final[:, 0])


# speedup vs baseline: 5.2484x; 1.2987x over previous
import functools

import jax
import jax.numpy as jnp
from jax import lax
from jax.experimental import pallas as pl
from jax.experimental.pallas import tpu as pltpu
from jax.experimental.pallas import tpu_sc as plsc

TOPK_K = 200
LANES = 128
REC = 8
RPR = LANES // REC
SEL = 224
GCH = 13


def _score_stage(det_ref, rec_ref):
    d = det_ref[0]
    C, F = d.shape
    dt = d.T
    row = lax.broadcasted_iota(jnp.int32, (F, C), 0)
    row_f = (row - 5).astype(jnp.float32)
    neg = jnp.float32(-jnp.inf)
    dm = jnp.where(row >= 5, dt, neg)
    m = jnp.max(dm, axis=0, keepdims=True)
    am = jnp.min(jnp.where(dm == m, row_f, jnp.float32(F)),
                 axis=0, keepdims=True)
    obj = jnp.max(jnp.where(row == 4, dt, neg), axis=0, keepdims=True)
    score = m * obj
    rec_ref[0] = jnp.concatenate(
        [dt[0:4, :], score, am, score, score], axis=0)


def _thresh_stage(sp_ref, thr_ref, *, k):
    s = sp_ref[...]
    B = s.shape[0]
    si = lax.bitcast_convert_type(s, jnp.int32)

    def body(i, t):
        bit = 30 - i
        tt = jnp.bitwise_or(t, lax.shift_left(jnp.int32(1), bit))
        cnt = jnp.sum((si >= tt).astype(jnp.int32), axis=1, keepdims=True)
        return jnp.where(cnt >= k, tt, t)

    t = lax.fori_loop(0, 31, body, jnp.zeros((B, 1), jnp.int32))
    thr_ref[...] = jnp.broadcast_to(t, thr_ref.shape)


def _tree16(vs, op):
    while len(vs) > 1:
        vs = [op(vs[i], vs[i + 1]) if i + 1 < len(vs) else vs[i]
              for i in range(0, len(vs), 2)]
    return vs[0]


def _select_gather(sp_hbm, thr_hbm, rec_hbm, out_hbm, sv, thr_v, gv_ref,
                   gi_ref, ei_ref, selrow, sub_v, ri1, ri2, rows1, rows2,
                   outv, sem, *, n, np_, k, nr, h, w):
    cid = lax.axis_index("c")
    sid = lax.axis_index("s")
    wid = sid * 2 + cid
    B = out_hbm.shape[0]
    lane = lax.broadcasted_iota(jnp.int32, (16,), 0)
    MIN = jnp.int32(-2**31)
    BIG = jnp.int32(1 << 30)

    @pl.when(wid < B)
    def _():
        b = wid
        pltpu.sync_copy(sp_hbm.at[b], sv)
        pltpu.sync_copy(thr_hbm.at[b], thr_v)
        t = thr_v[pl.ds(0, 16)][0]

        for i in range(SEL // 16):
            sl = pl.ds(i * 16, 16)
            gv_ref[sl] = jnp.zeros((16,), jnp.int32) + MIN
            gi_ref[sl] = jnp.zeros((16,), jnp.int32)

        def compact(i, carry):
            cg, ce = carry
            v = sv[pl.ds(i * 16, 16)]
            vi = plsc.bitcast(v, jnp.int32)
            idxv = i * 16 + lane
            m_g = vi > t
            m_e = vi == t
            plsc.store_compressed(gv_ref.at[pl.ds(cg, 16)], vi, mask=m_g)
            plsc.store_compressed(gi_ref.at[pl.ds(cg, 16)], idxv, mask=m_g)
            plsc.store_compressed(ei_ref.at[pl.ds(ce, 16)], idxv, mask=m_e)
            cg = cg + plsc.all_reduce_population_count(m_g)[0]
            ce = ce + plsc.all_reduce_population_count(m_e)[0]
            return cg, ce

        k1, _ = lax.fori_loop(0, np_ // 16, compact,
                              (jnp.int32(0), jnp.int32(0)))

        gvs = [gv_ref[pl.ds(i * 16, 16)] for i in range(GCH)]
        gis = [gi_ref[pl.ds(i * 16, 16)] for i in range(GCH)]

        def extract(j, carry):
            gvs, gis, acc = carry
            m = lax.reduce_max(_tree16(gvs, jnp.maximum), axes=(0,))
            cand = [jnp.where(gvs[i] == m, gis[i], BIG)
                    for i in range(GCH)]
            gi = lax.reduce_min(_tree16(cand, jnp.minimum), axes=(0,))
            gvs = [jnp.where((gvs[i] == m) & (gis[i] == gi), MIN, gvs[i])
                   for i in range(GCH)]
            acc = jnp.where(lane == jnp.bitwise_and(j, 15), gi, acc)
            selrow[pl.ds(jnp.bitwise_and(j, ~15), 16)] = acc
            return gvs, gis, acc

        lax.fori_loop(0, k, extract,
                      (gvs, gis, jnp.zeros((16,), jnp.int32)))

        for i in range(GCH):
            @pl.when(k1 + i * 16 < 208)
            def _():
                selrow[pl.ds(k1 + i * 16, 16)] = ei_ref[pl.ds(i * 16, 16)]

        for i in range(SEL // 16):
            sl = pl.ds(i * 16, 16)
            rv = selrow[sl]
            rv = jnp.minimum(jnp.maximum(rv, 0), n - 1)
            selrow[sl] = rv
            rid = b * nr + lax.shift_right_logical(rv, 4)
            sub_v[sl] = jnp.bitwise_and(rv, 15)
            if i < 8:
                ri1[pl.ds(i * 16, 16)] = rid
            else:
                ri2[pl.ds((i - 8) * 16, 16)] = rid

        cp1 = pltpu.async_copy(rec_hbm.at[ri1], rows1, sem)
        cp1.wait()
        cp2 = pltpu.async_copy(rec_hbm.at[ri2], rows2, sem)
        cp2.wait()

        b_f = b.astype(jnp.float32)
        bvec = jnp.zeros((16,), jnp.float32) + b_f

        def unpack(rows_v, t, pbase):
            sub = sub_v[pl.ds((t + pbase) * 16, 16)]
            rloc = t * 16 + lane

            def fld(c):
                return plsc.load_gather(rows_v, [rloc, c * 16 + sub])

            cx = fld(0)
            cy = fld(1)
            wv = fld(2)
            hv = fld(3)
            score = fld(4)
            cls = fld(5)
            hw = wv * jnp.float32(0.5)
            hh = hv * jnp.float32(0.5)
            vals = [
                bvec,
                cls,
                score,
                (cx - hw) * jnp.float32(1.0 / (h - 1.0)),
                (cy - hh) * jnp.float32(1.0 / (w - 1.0)),
                (cx + hw) * jnp.float32(1.0 / (h - 1.0)),
                (cy + hh) * jnp.float32(1.0 / (w - 1.0)),
            ]
            grow = (t + pbase) * 16 + lane
            for c2, v in enumerate(vals):
                colv = jnp.zeros((16,), jnp.int32) + c2
                plsc.store_scatter(outv, [grow, colv], v)

        def un1(t, _):
            unpack(rows1, t, 0)
            return 0

        def un2(t, _):
            unpack(rows2, t, 8)
            return 0

        lax.fori_loop(0, 8, un1, 0)
        lax.fori_loop(0, SEL // 16 - 8, un2, 0)
        pltpu.sync_copy(outv, out_hbm.at[b])


def kernel(x, det):
    B, N, F = det.shape
    H, W = x.shape[2], x.shape[3]
    K = TOPK_K
    NR = N // RPR
    C = 2000
    NC = N // C

    rec_t = pl.pallas_call(
        _score_stage,
        grid=(B,),
        in_specs=[pl.BlockSpec((1, N, F), lambda b: (b, 0, 0))],
        out_specs=pl.BlockSpec((1, REC, N), lambda b: (b, 0, 0)),
        out_shape=jax.ShapeDtypeStruct((B, REC, N), jnp.float32),
    )(det)

    rec = rec_t.reshape(B, REC, NR, RPR).transpose(0, 2, 1, 3)
    rec = rec.reshape(B * NR, LANES)

    n_pad = ((N + LANES - 1) // LANES) * LANES
    sp = jnp.pad(rec_t[:, 4, :], ((0, 0), (0, n_pad - N)),
                 constant_values=-jnp.inf)

    thr = pl.pallas_call(
        functools.partial(_thresh_stage, k=K),
        in_specs=[pl.BlockSpec((B, n_pad), lambda: (0, 0))],
        out_specs=pl.BlockSpec((B, LANES), lambda: (0, 0)),
        out_shape=jax.ShapeDtypeStruct((B, LANES), jnp.int32),
    )(sp)

    mesh = plsc.VectorSubcoreMesh(core_axis_name="c", subcore_axis_name="s")
    select_gather = pl.kernel(
        functools.partial(_select_gather, n=N, np_=n_pad, k=K, nr=NR,
                          h=float(H), w=float(W)),
        mesh=mesh,
        compiler_params=pltpu.CompilerParams(needs_layout_passes=False),
        out_type=jax.ShapeDtypeStruct((B, SEL, 16), jnp.float32),
        scratch_types=[
            pltpu.VMEM((n_pad,), jnp.float32),
            pltpu.VMEM((LANES,), jnp.int32),
            pltpu.VMEM((SEL,), jnp.int32),
            pltpu.VMEM((SEL,), jnp.int32),
            pltpu.VMEM((n_pad,), jnp.int32),
            pltpu.VMEM((SEL,), jnp.int32),
            pltpu.VMEM((SEL,), jnp.int32),
            pltpu.VMEM((128,), jnp.int32),
            pltpu.VMEM((96,), jnp.int32),
            pltpu.VMEM((128, LANES), jnp.float32),
            pltpu.VMEM((96, LANES), jnp.float32),
            pltpu.VMEM((SEL, 16), jnp.float32),
            pltpu.SemaphoreType.DMA,
        ],
    )
    tiles = select_gather(sp, thr, rec)

    final = tiles[:, :K, :7].reshape(B * K, 7)
    return (final, final[:, 0])
